# baseline (speedup 1.0000x reference)
import functools

import jax
import jax.numpy as jnp
from jax import lax
from jax.experimental import pallas as pl
from jax.experimental.pallas import tpu as pltpu
from jax.experimental.pallas import tpu_sc as plsc

N = 10000
E = 160000
DX = 256
DEN = 16
K = 16

NW = 32
NPN = 320
NPAD = NW * NPN

_mesh_cache = []


def _mesh():
    if not _mesh_cache:
        _mesh_cache.append(
            plsc.VectorSubcoreMesh(core_axis_name="c", subcore_axis_name="s"))
    return _mesh_cache[0]


def _wid():
    return lax.axis_index("s") * 2 + lax.axis_index("c")


_GCH = 128
_GNCH = E // _GCH


def _sc_gather_body(x_hbm, p_hbm, src_hbm, dst_hbm, xs_out, xd_out, ps_out,
                    sidx, didx, sbuf, dbuf, pbuf, sem1, sem2, sem3):
    w = _wid()
    trips = _GNCH // NW + jnp.where(w < _GNCH % NW, 1, 0)

    def body(t, _):
        c = w + t * NW
        off = c * _GCH
        pltpu.sync_copy(src_hbm.at[pl.ds(off, _GCH)], sidx)
        pltpu.sync_copy(dst_hbm.at[pl.ds(off, _GCH)], didx)
        cp1 = pltpu.async_copy(x_hbm.at[sidx], sbuf, sem1)
        cp2 = pltpu.async_copy(x_hbm.at[didx], dbuf, sem2)
        cp3 = pltpu.async_copy(p_hbm.at[sidx], pbuf, sem3)
        cp1.wait()
        cp2.wait()
        cp3.wait()
        pltpu.sync_copy(sbuf, xs_out.at[pl.ds(off, _GCH)])
        pltpu.sync_copy(dbuf, xd_out.at[pl.ds(off, _GCH)])
        pltpu.sync_copy(pbuf, ps_out.at[pl.ds(off, _GCH)])
        return 0

    lax.fori_loop(0, trips, body, 0)


def _sc_gather(x, p, src, dst):
    kfn = functools.partial(
        pl.kernel,
        mesh=_mesh(),
        out_type=[jax.ShapeDtypeStruct((E, DX), jnp.float32)] * 3,
        scratch_types=[
            pltpu.VMEM((_GCH,), jnp.int32),
            pltpu.VMEM((_GCH,), jnp.int32),
            pltpu.VMEM((_GCH, DX), jnp.float32),
            pltpu.VMEM((_GCH, DX), jnp.float32),
            pltpu.VMEM((_GCH, DX), jnp.float32),
            pltpu.SemaphoreType.DMA,
            pltpu.SemaphoreType.DMA,
            pltpu.SemaphoreType.DMA,
        ],
    )
    return kfn(_sc_gather_body)(x, p, src, dst)


def _tc_edge_body(xs_ref, xd_ref, ps_ref, w_ref, b_ref, out_ref):
    diff = (xd_ref[...] - xs_ref[...]).astype(jnp.bfloat16)
    t = jnp.dot(diff, w_ref[...],
                preferred_element_type=jnp.float32) + b_ref[...]
    out_ref[...] = t + ps_ref[...]


def _tc_edge_matmul(xs, xd, ps, w_bf, b):
    BR = 256
    grid = (E // BR,)
    return pl.pallas_call(
        _tc_edge_body,
        grid=grid,
        in_specs=[
            pl.BlockSpec((BR, DX), lambda i: (i, 0)),
            pl.BlockSpec((BR, DX), lambda i: (i, 0)),
            pl.BlockSpec((BR, DX), lambda i: (i, 0)),
            pl.BlockSpec((DX, DX), lambda i: (0, 0)),
            pl.BlockSpec((DX,), lambda i: (0,)),
        ],
        out_specs=pl.BlockSpec((BR, DX), lambda i: (i, 0)),
        out_shape=jax.ShapeDtypeStruct((E, DX), jnp.float32),
    )(xs, xd, ps, w_bf, b)


def _stage_a_body(x_ref, en_ref, wphi_ref, bphi_ref, wp1_ref, bp1_ref,
                  wp2_ref, bp2_ref, wp3_ref, bp3_ref, wp4_ref, bp4_ref,
                  wa1_ref, p_ref, pea_ref):
    xb = x_ref[...].astype(jnp.bfloat16)
    p_ref[...] = jnp.dot(xb, wphi_ref[...],
                         preferred_element_type=jnp.float32) + bphi_ref[...]
    en = en_ref[...]
    h = jnp.maximum(jnp.dot(en, wp1_ref[...]) + bp1_ref[...], 0.0)
    h = jnp.maximum(jnp.dot(h, wp2_ref[...]) + bp2_ref[...], 0.0)
    h = jnp.maximum(jnp.dot(h, wp3_ref[...]) + bp3_ref[...], 0.0)
    pe = jnp.dot(h, wp4_ref[...]) + bp4_ref[...]
    a = jnp.dot(en, wa1_ref[...])
    pea_ref[...] = jnp.concatenate(
        [pe, a, jnp.zeros((a.shape[0], 8), jnp.float32)], axis=1)


def _stage_a(x, en, wphi_bf, bphi, phi_pads, wa1):
    RP = 10240
    BR = 256
    xp = jnp.zeros((RP, DX), jnp.float32).at[:N].set(x)
    enp = jnp.zeros((RP, DEN), jnp.float32).at[:N].set(en)
    (wp1, bp1), (wp2, bp2), (wp3, bp3), (wp4, bp4) = phi_pads
    row = lambda i: (i, 0)
    full = lambda i: (0, 0)
    vec = lambda i: (0,)
    p, pea = pl.pallas_call(
        _stage_a_body,
        grid=(RP // BR,),
        in_specs=[
            pl.BlockSpec((BR, DX), row),
            pl.BlockSpec((BR, DEN), row),
            pl.BlockSpec((DX, DX), full),
            pl.BlockSpec((DX,), vec),
            pl.BlockSpec((16, 8), full), pl.BlockSpec((8,), vec),
            pl.BlockSpec((8, 8), full), pl.BlockSpec((8,), vec),
            pl.BlockSpec((8, 8), full), pl.BlockSpec((8,), vec),
            pl.BlockSpec((8, 16), full), pl.BlockSpec((16,), vec),
            pl.BlockSpec((16, 8), full),
        ],
        out_specs=[
            pl.BlockSpec((BR, DX), row),
            pl.BlockSpec((BR, 32), row),
        ],
        out_shape=[
            jax.ShapeDtypeStruct((RP, DX), jnp.float32),
            jax.ShapeDtypeStruct((RP, 32), jnp.float32),
        ],
    )(xp, enp, wphi_bf, bphi, wp1, bp1, wp2, bp2, wp3, bp3, wp4, bp4, wa1)
    return p[:N], pea[:N]


_BE = 256


def _tc_segmax_body(dst_ref, ev_ref, m_ref, cnt_ref, acc, cntv):
    i = pl.program_id(0)

    @pl.when(i == 0)
    def _():
        acc[...] = jnp.full((NPAD, DX), -jnp.inf, jnp.float32)
        cntv[...] = jnp.zeros((NPAD, 128), jnp.float32)

    one = jnp.ones((1, 128), jnp.float32)

    def body(e, _):
        d = dst_ref[e]
        row = ev_ref[pl.ds(e, 1), :]
        acc[pl.ds(d, 1), :] = jnp.maximum(acc[pl.ds(d, 1), :], row)
        cntv[pl.ds(d, 1), :] += one
        return 0

    lax.fori_loop(0, _BE, body, 0)

    @pl.when(i == pl.num_programs(0) - 1)
    def _():
        m_ref[...] = acc[...]
        cnt_ref[...] = cntv[...]


def _tc_segmax(dst, ev):
    m, cnt = pl.pallas_call(
        _tc_segmax_body,
        grid=(E // _BE,),
        in_specs=[
            pl.BlockSpec((_BE,), lambda i: (i,), memory_space=pltpu.SMEM),
            pl.BlockSpec((_BE, DX), lambda i: (i, 0)),
        ],
        out_specs=[
            pl.BlockSpec((NPAD, DX), lambda i: (0, 0)),
            pl.BlockSpec((NPAD, 128), lambda i: (0, 0)),
        ],
        out_shape=[
            jax.ShapeDtypeStruct((NPAD, DX), jnp.float32),
            jax.ShapeDtypeStruct((NPAD, 128), jnp.float32),
        ],
        scratch_shapes=[
            pltpu.VMEM((NPAD, DX), jnp.float32),
            pltpu.VMEM((NPAD, 128), jnp.float32),
        ],
    )(dst, ev)
    return m, cnt


def _mlp_pad(params):
    dims = [16, 8, 8, 8, 16]
    out = []
    for i, (W, b) in enumerate(params):
        dw_in, dw_out = W.shape
        pi, po = dims[i], dims[i + 1]
        Wp = jnp.zeros((pi, po), jnp.float32).at[:dw_in, :dw_out].set(W)
        bp = jnp.zeros((po,), jnp.float32).at[:dw_out].set(b)
        out.append((Wp, bp))
    return out


def kernel(x, en, edge_index, theta_W, theta_b, phi_W, phi_b,
           theta_en_params, phi_en_params):
    src = edge_index[0].astype(jnp.int32)
    dst = edge_index[1].astype(jnp.int32)

    phi_pads = _mlp_pad(phi_en_params)
    theta_pads = _mlp_pad(theta_en_params)
    wa1 = theta_pads[0][0]

    p, pea = _stage_a(x, en, phi_W.astype(jnp.bfloat16), phi_b, phi_pads, wa1)
    pe = pea[:, :16]
    a = pea[:, 16:24]

    xs, xd, ps = _sc_gather(x, p, src, dst)
    ev = _tc_edge_matmul(xs, xd, ps, theta_W.astype(jnp.bfloat16), theta_b)
    m, cntv = _tc_segmax(dst, ev)
    m = m[:N]
    cnt = cntv[:N, 0]
    x_new = jnp.where((cnt > 0)[:, None], m, 0.0)

    b1 = theta_pads[0][1]
    h = jnp.maximum(jnp.take(a, dst, axis=0) - jnp.take(a, src, axis=0) + b1,
                    0.0)
    for (W, b) in theta_pads[1:]:
        h = jnp.dot(h, W) + b
        if W.shape[1] != 16:
            h = jnp.maximum(h, 0.0)
    msg = h + jnp.take(pe, src, axis=0)
    en_new = jax.ops.segment_sum(msg, dst, num_segments=N) / jnp.maximum(
        cnt, 1.0)[:, None]

    sq = jnp.sum(x_new * x_new, axis=1)
    d2 = sq[:, None] + sq[None, :] - 2.0 * (x_new @ x_new.T)
    _, knn_idx = jax.lax.top_k(-d2, K)
    src_new = knn_idx.reshape(-1)
    dst_new = jnp.repeat(jnp.arange(N, dtype=knn_idx.dtype), K)
    knn_edge_index = jnp.stack([src_new, dst_new], axis=0)
    return (knn_edge_index, x_new, en_new)
